# baseline (speedup 1.0000x reference)
import functools

import jax
import jax.numpy as jnp
from jax import lax
from jax.experimental import pallas as pl
from jax.experimental.pallas import tpu as pltpu
from jax.experimental.pallas import tpu_sc as plsc

EMBED = 64
BATCH = 16384

_info = plsc.get_sparse_core_info()
_NC = _info.num_cores
_NS = _info.num_subcores
_NW = _NC * _NS
_BPW = BATCH // _NW
_CH = 128
_NCHUNK = _BPW // _CH

_mesh = plsc.VectorSubcoreMesh(core_axis_name="c", subcore_axis_name="s")


@functools.partial(
    pl.kernel,
    mesh=_mesh,
    out_type=jax.ShapeDtypeStruct((BATCH, 2, EMBED), jnp.float32),
    scratch_types=[
        pltpu.VMEM((_NCHUNK, _CH), jnp.int32),
        pltpu.VMEM((_NCHUNK, _CH), jnp.int32),
        pltpu.VMEM((_BPW, EMBED), jnp.float32),
        pltpu.VMEM((_BPW, EMBED), jnp.float32),
        pltpu.SemaphoreType.DMA,
    ],
    compiler_params=pltpu.CompilerParams(use_tc_tiling_on_sc=False),
)
def _embed_gather(ids_hbm, wu_hbm, wm_hbm, out_hbm,
                  idx_u, idx_m, rows_u, rows_m, sem):
    wid = lax.axis_index("s") * _NC + lax.axis_index("c")
    base = wid * _BPW

    for j in range(_NCHUNK):
        pltpu.sync_copy(ids_hbm.at[0, pl.ds(base + j * _CH, _CH)], idx_u.at[j])
        pltpu.sync_copy(ids_hbm.at[1, pl.ds(base + j * _CH, _CH)], idx_m.at[j])

    copies = []
    for j in range(_NCHUNK):
        copies.append(pltpu.async_copy(
            wu_hbm.at[idx_u.at[j]], rows_u.at[pl.ds(j * _CH, _CH)], sem))
        copies.append(pltpu.async_copy(
            wm_hbm.at[idx_m.at[j]], rows_m.at[pl.ds(j * _CH, _CH)], sem))
    for c in copies:
        c.wait()

    out_u = pltpu.async_copy(rows_u, out_hbm.at[pl.ds(base, _BPW), 0], sem)
    out_m = pltpu.async_copy(rows_m, out_hbm.at[pl.ds(base, _BPW), 1], sem)
    out_u.wait()
    out_m.wait()


def kernel(input, W_user, W_movie):
    out = _embed_gather(input, W_user, W_movie)
    return out.reshape(BATCH, 2 * EMBED)


# speedup vs baseline: 1.8275x; 1.8275x over previous
import functools

import jax
import jax.numpy as jnp
from jax import lax
from jax.experimental import pallas as pl
from jax.experimental.pallas import tpu as pltpu
from jax.experimental.pallas import tpu_sc as plsc

EMBED = 64
BATCH = 16384

_info = plsc.get_sparse_core_info()
_NC = _info.num_cores
_NS = _info.num_subcores
_NW = _NC * _NS
_BPW = BATCH // _NW

_mesh = plsc.VectorSubcoreMesh(core_axis_name="c", subcore_axis_name="s")


@functools.partial(
    pl.kernel,
    mesh=_mesh,
    out_type=jax.ShapeDtypeStruct((BATCH, 2 * EMBED), jnp.float32),
    scratch_types=[
        pltpu.VMEM((_BPW,), jnp.int32),
        pltpu.VMEM((_BPW,), jnp.int32),
        pltpu.VMEM((_BPW, 2 * EMBED), jnp.float32),
        pltpu.SemaphoreType.DMA,
    ],
)
def _embed_gather(ids_hbm, wu_hbm, wm_hbm, out_hbm,
                  idx_u, idx_m, combined, sem):
    wid = lax.axis_index("s") * _NC + lax.axis_index("c")
    base = wid * _BPW

    pltpu.sync_copy(ids_hbm.at[0, pl.ds(base, _BPW)], idx_u)
    pltpu.sync_copy(ids_hbm.at[1, pl.ds(base, _BPW)], idx_m)

    def issue_group(g, carry):
        vu = idx_u[pl.ds(g * 16, 16)]
        vm = idx_m[pl.ds(g * 16, 16)]
        for lane in range(16):
            j = g * 16 + lane
            pltpu.async_copy(wu_hbm.at[vu[lane]],
                             combined.at[j, pl.ds(0, EMBED)], sem)
            pltpu.async_copy(wm_hbm.at[vm[lane]],
                             combined.at[j, pl.ds(EMBED, EMBED)], sem)
        return carry

    lax.fori_loop(0, _BPW // 16, issue_group, 0)

    pltpu.make_async_copy(out_hbm.at[pl.ds(0, _BPW), :], combined, sem).wait()

    pltpu.sync_copy(combined, out_hbm.at[pl.ds(base, _BPW), :])


def kernel(input, W_user, W_movie):
    return _embed_gather(input, W_user, W_movie)


# speedup vs baseline: 1.8286x; 1.0006x over previous
import functools

import jax
import jax.numpy as jnp
from jax import lax
from jax.experimental import pallas as pl
from jax.experimental.pallas import tpu as pltpu
from jax.experimental.pallas import tpu_sc as plsc

EMBED = 64
BATCH = 16384

_info = plsc.get_sparse_core_info()
_NC = _info.num_cores
_NS = _info.num_subcores
_NW = _NC * _NS
_BPW = BATCH // _NW

_mesh = plsc.VectorSubcoreMesh(core_axis_name="c", subcore_axis_name="s")


@functools.partial(
    pl.kernel,
    mesh=_mesh,
    out_type=jax.ShapeDtypeStruct((BATCH, 2 * EMBED), jnp.float32),
    scratch_types=[
        pltpu.VMEM((_BPW,), jnp.int32),
        pltpu.VMEM((_BPW,), jnp.int32),
        pltpu.VMEM((_BPW, 2 * EMBED), jnp.float32),
        pltpu.SemaphoreType.DMA,
    ],
)
def _embed_gather(ids_hbm, wu_hbm, wm_hbm, out_hbm,
                  idx_u, idx_m, combined, sem):
    wid = lax.axis_index("s") * _NC + lax.axis_index("c")
    base = wid * _BPW

    pltpu.sync_copy(ids_hbm.at[0, pl.ds(base, _BPW)], idx_u)
    pltpu.sync_copy(ids_hbm.at[1, pl.ds(base, _BPW)], idx_m)

    @plsc.parallel_loop(0, _BPW // 16, 1, unroll=2)
    def _issue(g):
        vu = idx_u[pl.ds(g * 16, 16)]
        vm = idx_m[pl.ds(g * 16, 16)]
        for lane in range(16):
            j = g * 16 + lane
            pltpu.async_copy(wu_hbm.at[vu[lane]],
                             combined.at[j, pl.ds(0, EMBED)], sem)
            pltpu.async_copy(wm_hbm.at[vm[lane]],
                             combined.at[j, pl.ds(EMBED, EMBED)], sem)

    pltpu.make_async_copy(out_hbm.at[pl.ds(0, _BPW), :], combined, sem).wait()

    pltpu.sync_copy(combined, out_hbm.at[pl.ds(base, _BPW), :])


def kernel(input, W_user, W_movie):
    return _embed_gather(input, W_user, W_movie)
